# baseline (speedup 1.0000x reference)
import functools

import jax
import jax.numpy as jnp
from jax import lax
from jax.experimental import pallas as pl
from jax.experimental.pallas import tpu as pltpu
from jax.experimental.pallas import tpu_sc as plsc

NC, NS, L = 2, 16, 16
NW = NC * NS
B, SEQ, D = 1024, 200, 128
BPW = B // NW
CHA, CHB = 104, 96
ND = D // L

_mesh = plsc.VectorSubcoreMesh(
    core_axis_name="c", subcore_axis_name="s", num_cores=NC, num_subcores=NS
)


@functools.partial(
    pl.kernel,
    out_type=jax.ShapeDtypeStruct((B, D), jnp.float32),
    mesh=_mesh,
    scratch_types=[
        pltpu.VMEM((BPW * SEQ,), jnp.int32),
        pltpu.VMEM((CHA, D), jnp.float32),
        pltpu.VMEM((CHB, D), jnp.float32),
        pltpu.VMEM((CHA, D), jnp.float32),
        pltpu.VMEM((CHB, D), jnp.float32),
        pltpu.VMEM((CHA, D), jnp.float32),
        pltpu.VMEM((CHB, D), jnp.float32),
        pltpu.VMEM((CHA, D), jnp.float32),
        pltpu.VMEM((CHB, D), jnp.float32),
        pltpu.VMEM((BPW, D), jnp.float32),
        pltpu.SemaphoreType.DMA,
        pltpu.SemaphoreType.DMA,
        pltpu.SemaphoreType.DMA,
        pltpu.SemaphoreType.DMA,
        pltpu.SemaphoreType.DMA,
        pltpu.SemaphoreType.DMA,
        pltpu.SemaphoreType.DMA,
        pltpu.SemaphoreType.DMA,
    ],
)
def _encode(x_hbm, table_hbm, out_hbm, idx_v, ra0, rb0, ra1, rb1,
            ra2, rb2, ra3, rb3, out_v,
            sa0, sb0, sa1, sb1, sa2, sb2, sa3, sb3):
    wid = lax.axis_index("s") * NC + lax.axis_index("c")

    pltpu.sync_copy(x_hbm.at[pl.ds(wid * BPW * SEQ, BPW * SEQ)], idx_v)

    zero = tuple(jnp.zeros((L,), jnp.float32) for _ in range(ND))
    ras, rbs = (ra0, ra1, ra2, ra3), (rb0, rb1, rb2, rb3)
    sas, sbs = (sa0, sa1, sa2, sa3), (sb0, sb1, sb2, sb3)

    def ia(s):
        return idx_v.at[pl.ds(pl.multiple_of(s * SEQ, 8), CHA)]

    def ib(s):
        return idx_v.at[pl.ds(pl.multiple_of(s * SEQ + CHA, 8), CHB)]

    def fire_a(s, h):
        pltpu.async_copy(table_hbm.at[ia(s)], ras[h], sas[h])

    def fire_b(s, h):
        pltpu.async_copy(table_hbm.at[ib(s)], rbs[h], sbs[h])

    def wait_a(s, h):
        pltpu.make_async_copy(table_hbm.at[ia(s)], ras[h], sas[h]).wait()

    def wait_b(s, h):
        pltpu.make_async_copy(table_hbm.at[ib(s)], rbs[h], sbs[h]).wait()

    def reduce_chunk(buf, n, acc):
        def row_body(r, a):
            return tuple(a[j] + buf[r, pl.ds(j * L, L)] for j in range(ND))

        return lax.fori_loop(0, n, row_body, acc)

    for h in range(4):
        fire_a(h, h)
        fire_b(h, h)

    def quad_body(i, carry):
        for h in range(4):
            s = 4 * i + h
            wait_a(s, h)
            acc = reduce_chunk(ras[h], CHA, zero)

            @pl.when(s + 4 < BPW)
            def _():
                fire_a(s + 4, h)

            wait_b(s, h)
            acc = reduce_chunk(rbs[h], CHB, acc)

            @pl.when(s + 4 < BPW)
            def _():
                fire_b(s + 4, h)

            for j in range(ND):
                out_v[s, pl.ds(j * L, L)] = acc[j]
        return carry

    lax.fori_loop(0, BPW // 4, quad_body, 0)
    pltpu.sync_copy(out_v, out_hbm.at[pl.ds(wid * BPW, BPW)])


def kernel(x, table):
    return _encode(x.astype(jnp.int32).reshape(-1), table)


# speedup vs baseline: 1.0334x; 1.0334x over previous
import functools

import jax
import jax.numpy as jnp
from jax import lax
from jax.experimental import pallas as pl
from jax.experimental.pallas import tpu as pltpu
from jax.experimental.pallas import tpu_sc as plsc

NC, NS, L = 2, 16, 16
NW = NC * NS
B, SEQ, D = 1024, 200, 128
BPW = B // NW
CHA, CHB = 104, 96
ND = D // L

_mesh = plsc.VectorSubcoreMesh(
    core_axis_name="c", subcore_axis_name="s", num_cores=NC, num_subcores=NS
)


@functools.partial(
    pl.kernel,
    out_type=jax.ShapeDtypeStruct((B, D), jnp.float32),
    mesh=_mesh,
    scratch_types=[
        pltpu.VMEM((BPW * SEQ,), jnp.int32),
        pltpu.VMEM((CHA, D), jnp.float32),
        pltpu.VMEM((CHB, D), jnp.float32),
        pltpu.VMEM((CHA, D), jnp.float32),
        pltpu.VMEM((CHB, D), jnp.float32),
        pltpu.VMEM((BPW, D), jnp.float32),
        pltpu.SemaphoreType.DMA,
        pltpu.SemaphoreType.DMA,
        pltpu.SemaphoreType.DMA,
        pltpu.SemaphoreType.DMA,
    ],
)
def _encode(x_hbm, table_hbm, out_hbm, idx_v, ra0, rb0, ra1, rb1, out_v,
            sa0, sb0, sa1, sb1):
    wid = lax.axis_index("s") * NC + lax.axis_index("c")

    pltpu.sync_copy(x_hbm.at[pl.ds(wid * BPW * SEQ, BPW * SEQ)], idx_v)

    zero = tuple(jnp.zeros((L,), jnp.float32) for _ in range(ND))
    ras, rbs = (ra0, ra1), (rb0, rb1)
    sas, sbs = (sa0, sa1), (sb0, sb1)

    def ia(s):
        return idx_v.at[pl.ds(pl.multiple_of(s * SEQ, 8), CHA)]

    def ib(s):
        return idx_v.at[pl.ds(pl.multiple_of(s * SEQ + CHA, 8), CHB)]

    def fire_a(s, h):
        pltpu.async_copy(table_hbm.at[ia(s)], ras[h], sas[h])

    def fire_b(s, h):
        pltpu.async_copy(table_hbm.at[ib(s)], rbs[h], sbs[h])

    def wait_a(s, h):
        pltpu.make_async_copy(table_hbm.at[ia(s)], ras[h], sas[h]).wait()

    def wait_b(s, h):
        pltpu.make_async_copy(table_hbm.at[ib(s)], rbs[h], sbs[h]).wait()

    def reduce_chunk(buf, n, acc):
        def row_body(r, a):
            return tuple(a[j] + buf[r, pl.ds(j * L, L)] for j in range(ND))

        return lax.fori_loop(0, n, row_body, acc)

    fire_a(0, 0)
    fire_b(0, 0)
    fire_a(1, 1)
    fire_b(1, 1)

    def pair_body(i, carry):
        for h in range(2):
            s = 2 * i + h
            wait_a(s, h)
            acc = reduce_chunk(ras[h], CHA, zero)

            @pl.when(s + 2 < BPW)
            def _():
                fire_a(s + 2, h)

            wait_b(s, h)
            acc = reduce_chunk(rbs[h], CHB, acc)

            @pl.when(s + 2 < BPW)
            def _():
                fire_b(s + 2, h)

            for j in range(ND):
                out_v[s, pl.ds(j * L, L)] = acc[j]
        return carry

    lax.fori_loop(0, BPW // 2, pair_body, 0)
    pltpu.sync_copy(out_v, out_hbm.at[pl.ds(wid * BPW, BPW)])


def kernel(x, table):
    return _encode(x.astype(jnp.int32).reshape(-1), table)
